# Initial kernel scaffold; baseline (speedup 1.0000x reference)
import jax
import jax.numpy as jnp
from jax.experimental import pallas as pl


def kernel(boxes, scores, labels):
    raise NotImplementedError("write your pallas kernel here")


# speedup vs baseline: 49.2399x; 49.2399x over previous
import functools

import jax
import jax.numpy as jnp
from jax import lax
from jax.experimental import pallas as pl
from jax.experimental.pallas import tpu as pltpu
from jax.experimental.pallas import tpu_sc as plsc

_N = 5000
_NP = 5120
_NCHUNK = _NP // 128
_NCLS = 81
_SCORE_THRESH = 0.2
_NMS_THRESH = 0.5
_MAXH = 15
_MAXO = 15


def _rank_kernel(kcol_ref, krow_ref, boxesT_ref, labelsf_ref, scrow_ref,
                 sidx_ref, rows_ref):
    kcol = kcol_ref[...]
    kcol = jnp.where(kcol >= _SCORE_THRESH, kcol, -1.0)
    idx_col = lax.broadcasted_iota(jnp.int32, (_NP, 1), 0)

    def rank_body(jr, rank):
        krow = krow_ref[pl.ds(jr, 1), :]
        krow = jnp.where(krow >= _SCORE_THRESH, krow, -1.0)
        idx_row = jr * 128 + lax.broadcasted_iota(jnp.int32, (1, 128), 1)
        before = (krow > kcol) | ((krow == kcol) & (idx_row < idx_col))
        return rank + jnp.sum(before.astype(jnp.int32), axis=1, keepdims=True)

    rank = lax.fori_loop(0, _NCHUNK, rank_body,
                         jnp.zeros((_NP, 1), jnp.int32))

    def inv_body(pr, carry):
        p_row = pr * 128 + lax.broadcasted_iota(jnp.int32, (1, 128), 1)
        hit = rank == p_row
        sidx_ref[pl.ds(pr, 1), :] = jnp.sum(
            jnp.where(hit, idx_col, 0), axis=0, keepdims=True)
        return carry

    lax.fori_loop(0, _NCHUNK, inv_body, 0)
    bt = boxesT_ref[...]
    b4 = bt[0:4, :]
    mx = jnp.max(bt)
    l4 = labelsf_ref[0:4, :]
    bnms4 = b4 + l4 * (mx + 1.0)
    rows_ref[...] = jnp.concatenate(
        [bnms4, b4, scrow_ref[...], labelsf_ref[0:1, :],
         jnp.zeros((6, _NP), jnp.float32)], axis=0)


def _sc_scan(sidx_hbm, rows_hbm, out_hbm,
             sidx_v, fld_v, srt_v, kx1, ky1, kx2, ky2, kar, cnt_v, out_v):
    cid = lax.axis_index("c")
    sid = lax.axis_index("s")
    lane16 = jnp.arange(16, dtype=jnp.int32)

    @pl.when((cid == 0) & (sid == 0))
    def _work():
        pltpu.sync_copy(sidx_hbm, sidx_v)
        for sr, r in enumerate((0, 1, 2, 3, 8, 9)):
            pltpu.sync_copy(rows_hbm.at[r], fld_v.at[sr])

        zf = jnp.zeros((16,), jnp.float32)
        zi = jnp.zeros((16,), jnp.int32)

        def presort(t, carry):
            idxv = sidx_v[pl.ds(t * 16, 16)]
            for sr in range(6):
                rowv = jnp.full((16,), sr, jnp.int32)
                srt_v[sr, pl.ds(t * 16, 16)] = plsc.load_gather(
                    fld_v, [rowv, idxv])
            return carry

        lax.fori_loop(0, _NP // 16, presort, 0)

        for sr in range(4):
            pltpu.sync_copy(rows_hbm.at[4 + sr], fld_v.at[sr])

        def zero_kept(t, carry):
            kx1[pl.ds(t * 16, 16)] = zf
            ky1[pl.ds(t * 16, 16)] = zf
            kx2[pl.ds(t * 16, 16)] = zf
            ky2[pl.ds(t * 16, 16)] = zf
            kar[pl.ds(t * 16, 16)] = zf
            return carry

        lax.fori_loop(0, _NCLS, zero_kept, 0)

        def zero_cnt(t, carry):
            cnt_v[pl.ds(t * 16, 16)] = zi
            return carry

        lax.fori_loop(0, _NCLS + 2, zero_cnt, 0)

        def zero_out(t, carry):
            out_v[pl.ds(t * 16, 16)] = zf
            return carry

        lax.fori_loop(0, 32, zero_out, 0)

        emit_rows = jnp.where(lane16 < 4, lane16, 0)

        def group_body(g, carry):
            done = cnt_v[pl.ds(82 * 16, 16)][0]

            @pl.when(done == 0)
            def _block():
                b = g * 16
                wx1 = srt_v[0, pl.ds(b, 16)]
                wy1 = srt_v[1, pl.ds(b, 16)]
                wx2 = srt_v[2, pl.ds(b, 16)]
                wy2 = srt_v[3, pl.ds(b, 16)]
                wsc = srt_v[4, pl.ds(b, 16)]
                wlb = srt_v[5, pl.ds(b, 16)]
                wsi = sidx_v[pl.ds(b, 16)]
                vtrue = lane16 >= 0
                dnums = lax.GatherDimensionNumbers(
                    offset_dims=(), collapsed_slice_dims=(0,),
                    start_index_map=(0,))

                def lane_of(v, tv):
                    return lax.gather(
                        v, tv[:, None], dnums, (1,),
                        mode=lax.GatherScatterMode.PROMISE_IN_BOUNDS)[0]

                def step(t, d):
                    tv = zi + t
                    s = lane_of(wsc, tv)
                    c = lane_of(wlb, tv).astype(jnp.int32)
                    valid = s >= _SCORE_THRESH
                    cidx = c * 16 + lane16
                    cc = plsc.load_gather(cnt_v, [cidx])[0]
                    oc = cnt_v[pl.ds(81 * 16, 16)][0]
                    is_h = c == 0
                    cap = (cc < _MAXH) & (is_h | (oc < _MAXO))
                    proceed = (d == 0) & valid & cap
                    cx1 = lane_of(wx1, tv)
                    cy1 = lane_of(wy1, tv)
                    cx2 = lane_of(wx2, tv)
                    cy2 = lane_of(wy2, tv)
                    car = (cx2 - cx1) * (cy2 - cy1)
                    vx1 = plsc.load_gather(kx1, [cidx])
                    vy1 = plsc.load_gather(ky1, [cidx])
                    vx2 = plsc.load_gather(kx2, [cidx])
                    vy2 = plsc.load_gather(ky2, [cidx])
                    var = plsc.load_gather(kar, [cidx])
                    ltx = jnp.maximum(vx1, cx1)
                    lty = jnp.maximum(vy1, cy1)
                    rbx = jnp.minimum(vx2, cx2)
                    rby = jnp.minimum(vy2, cy2)
                    w = jnp.maximum(rbx - ltx, 0.0)
                    h = jnp.maximum(rby - lty, 0.0)
                    inter = w * h
                    union = (var + car) - inter
                    iou = inter / jnp.maximum(union, 1e-9)
                    supp = (iou > _NMS_THRESH) & (lane16 < cc)
                    suppressed = plsc.all_reduce_population_count(supp)[0] > 0
                    keep = proceed & jnp.logical_not(suppressed)
                    kv = vtrue & keep
                    lcc = (lane16 == cc) & kv
                    plsc.store_scatter(kx1, [cidx], zf + cx1, mask=lcc)
                    plsc.store_scatter(ky1, [cidx], zf + cy1, mask=lcc)
                    plsc.store_scatter(kx2, [cidx], zf + cx2, mask=lcc)
                    plsc.store_scatter(ky2, [cidx], zf + cy2, mask=lcc)
                    plsc.store_scatter(kar, [cidx], zf + car, mask=lcc)
                    l0k = (lane16 == 0) & kv
                    plsc.store_scatter(cnt_v, [cidx], zi + cc + 1, mask=l0k)
                    row = jnp.where(is_h, cc, 15 + oc)
                    si = lane_of_i(wsi, tv)
                    bo4 = plsc.load_gather(fld_v, [emit_rows, zi + si])
                    vals = jnp.where(
                        lane16 == 0, bo4[0],
                        jnp.where(lane16 == 1, bo4[1],
                                  jnp.where(lane16 == 2, bo4[2],
                                            jnp.where(lane16 == 3, bo4[3],
                                                      jnp.where(lane16 == 4,
                                                                s, 0.0)))))
                    plsc.store_scatter(out_v, [row * 16 + lane16], vals,
                                       mask=kv)
                    plsc.store_scatter(cnt_v, [81 * 16 + lane16], zi + oc + 1,
                                       mask=l0k & jnp.logical_not(is_h))
                    hc2 = cnt_v[pl.ds(0, 16)][0]
                    oc2 = cnt_v[pl.ds(81 * 16, 16)][0]
                    stop = jnp.logical_not(valid)
                    stop = stop | ((hc2 >= _MAXH) & (oc2 >= _MAXO))
                    return jnp.where(stop, jnp.int32(1), d)

                def lane_of_i(v, tv):
                    return lax.gather(
                        v, tv[:, None], dnums, (1,),
                        mode=lax.GatherScatterMode.PROMISE_IN_BOUNDS)[0]

                d = lax.fori_loop(0, 16, step, jnp.int32(0))
                cnt_v[pl.ds(82 * 16, 16)] = jnp.where(lane16 == 0, d, zi)

            return carry

        lax.fori_loop(0, _NP // 16, group_body, 0)
        pltpu.sync_copy(out_v, out_hbm)


@functools.cache
def _build_sc_scan():
    mesh = plsc.VectorSubcoreMesh(core_axis_name="c", subcore_axis_name="s")
    return pl.kernel(
        _sc_scan,
        out_type=jax.ShapeDtypeStruct((512,), jnp.float32),
        mesh=mesh,
        compiler_params=pltpu.CompilerParams(needs_layout_passes=False),
        scratch_types=[
            pltpu.VMEM((_NP,), jnp.int32),
            pltpu.VMEM((6, _NP), jnp.float32),
            pltpu.VMEM((6, _NP), jnp.float32),
            pltpu.VMEM((_NCLS * 16,), jnp.float32),
            pltpu.VMEM((_NCLS * 16,), jnp.float32),
            pltpu.VMEM((_NCLS * 16,), jnp.float32),
            pltpu.VMEM((_NCLS * 16,), jnp.float32),
            pltpu.VMEM((_NCLS * 16,), jnp.float32),
            pltpu.VMEM(((_NCLS + 2) * 16,), jnp.int32),
            pltpu.VMEM((512,), jnp.float32),
        ],
    )


def kernel(boxes, scores, labels):
    n = boxes.shape[0]
    pad = _NP - n
    f32 = jnp.float32
    scores_p = jnp.concatenate([scores.astype(f32),
                                jnp.full((pad,), -1.0, f32)])
    labels_p = jnp.concatenate([labels.astype(jnp.int32),
                                jnp.zeros((pad,), jnp.int32)])
    boxes_p = jnp.concatenate([boxes.astype(f32),
                               jnp.zeros((pad, 4), f32)], axis=0)
    boxesT = jnp.concatenate([boxes_p.T, jnp.zeros((4, _NP), f32)], axis=0)
    labelsf = jnp.broadcast_to(labels_p.astype(f32)[None, :], (8, _NP))
    kcol = scores_p.reshape(_NP, 1)
    krow = scores_p.reshape(_NCHUNK, 128)
    scrow = scores_p.reshape(1, _NP)

    sidx2d, rows16T = pl.pallas_call(
        _rank_kernel,
        out_shape=[
            jax.ShapeDtypeStruct((_NCHUNK, 128), jnp.int32),
            jax.ShapeDtypeStruct((16, _NP), jnp.float32),
        ],
    )(kcol, krow, boxesT, labelsf, scrow)

    out512 = _build_sc_scan()(sidx2d.reshape(_NP), rows16T)
    return out512[:480].reshape(30, 16)[:, :5]
